# baseline (speedup 1.0000x reference)
import functools

import jax
import jax.numpy as jnp
from jax import lax
from jax.experimental import pallas as pl
from jax.experimental.pallas import tpu as pltpu
from jax.experimental.pallas import tpu_sc as plsc

N = 10000
E = 320000
D_NODE = 128
D_EDGE = 16
W = 64
W2 = 2 * W

NC = 2
NS = 16
NW = NC * NS

HALF = E // 2
HP = 163840
TILE_N = 1000
TILE_P = 1000
N_PAD = 10240
CH = 64
K = 8

PPW = HP // NW
P_CH = PPW // CH
P_BLK = P_CH // K
NBLK = NW * P_BLK
B_C0 = 15
B_C1 = (NBLK - NS * B_C0) // NS
NPT = N_PAD // NS
NIB = HP // CH


@functools.lru_cache(maxsize=None)
def _sc_mesh():
    return plsc.VectorSubcoreMesh(
        core_axis_name="c", subcore_axis_name="s",
        num_cores=NC, num_subcores=NS)


def _dot(x, w):
    return lax.dot_general(x, w, (((1,), (0,)), ((), ())),
                           preferred_element_type=jnp.float32)


def _mlp(x, wi, bi, wa, ba, wb, bb, wo, bo):
    h = jnp.maximum(_dot(x, wi) + bi, 0.0)
    h1 = jnp.maximum(_dot(h, wa) + ba, 0.0)
    h2 = jnp.maximum(_dot(h1, wb) + bb, 0.0)
    h = h + h2
    return _dot(h, wo) + bo


def _full(shape):
    return pl.BlockSpec(shape, lambda i: (0,) * len(shape))


def _dotb(x, w):
    return lax.dot_general(x.astype(jnp.bfloat16), w.astype(jnp.bfloat16),
                           (((1,), (0,)), ((), ())),
                           preferred_element_type=jnp.float32)


def _mlpb(x, wi, bi, wa, ba, wb, bb, wo, bo):
    h = jnp.maximum(_dotb(x, wi) + bi, 0.0)
    h1 = jnp.maximum(_dotb(h, wa) + ba, 0.0)
    h2 = jnp.maximum(_dotb(h1, wb) + bb, 0.0)
    h = h + h2
    return _dotb(h, wo) + bo


def _node_enc_body(nf, wi, bi, wa, ba, wb, bb, wo, bo, pw2, pw3,
                   out, outq0, outq1):
    ne = _mlp(nf[...], wi[...], bi[...], wa[...], ba[...],
              wb[...], bb[...], wo[...], bo[...])
    out[...] = ne
    outq0[...] = _dot(ne, pw2[...])
    outq1[...] = _dot(ne, pw3[...])


def _node_encoder(nf, ws):
    in_specs = [pl.BlockSpec((TILE_N, D_NODE), lambda i: (i, 0))]
    in_specs += [_full(w.shape) for w in ws]
    ospec = pl.BlockSpec((TILE_N, W), lambda i: (i, 0))
    oshape = jax.ShapeDtypeStruct((N, W), jnp.float32)
    return pl.pallas_call(
        _node_enc_body,
        grid=(N // TILE_N,),
        in_specs=in_specs,
        out_specs=[ospec, ospec, ospec],
        out_shape=[oshape, oshape, oshape],
    )(nf, *ws)


def _sc_gather_body(q0, q1, i0a, i0b, i1a, i1b, out_hbm,
                    i0a_v, i0b_v, i1a_v, i1b_v, rowsa_v, rowsb_v,
                    sem_g, sem_a, sem_s):
    cid = lax.axis_index("c")
    sid = lax.axis_index("s")

    @pl.when(cid == 0)
    def _():
        _gather_loop(q0, q1, i0a, i0b, i1a, i1b, out_hbm, i0a_v, i0b_v,
                     i1a_v, i1b_v, rowsa_v, rowsb_v, sem_g, sem_a, sem_s,
                     sid * B_C0, B_C0)

    @pl.when(cid == 1)
    def _():
        _gather_loop(q0, q1, i0a, i0b, i1a, i1b, out_hbm, i0a_v, i0b_v,
                     i1a_v, i1b_v, rowsa_v, rowsb_v, sem_g, sem_a, sem_s,
                     NS * B_C0 + sid * B_C1, B_C1)


def _gather_loop(q0, q1, i0a, i0b, i1a, i1b, out_hbm,
                 i0a_v, i0b_v, i1a_v, i1b_v, rowsa_v, rowsb_v,
                 sem_g, sem_a, sem_s, blk0, nblk):
    @pl.loop(0, nblk)
    def _(g):
        row0 = (blk0 + g) * K
        start = row0 * CH
        pltpu.sync_copy(i0a.at[pl.ds(row0, K)], i0a_v)
        pltpu.sync_copy(i0b.at[pl.ds(row0, K)], i0b_v)
        pltpu.sync_copy(i1a.at[pl.ds(row0, K)], i1a_v)
        pltpu.sync_copy(i1b.at[pl.ds(row0, K)], i1b_v)
        gs = []
        for b in range(K):
            gs.append(pltpu.async_copy(
                q0.at[i0a_v.at[b]], rowsa_v.at[b], sem_g))
            gs.append(pltpu.async_copy(
                q0.at[i0b_v.at[b]], rowsb_v.at[b], sem_g))
        ads = []
        for b in range(K):
            gs[2 * b].wait()
            gs[2 * b + 1].wait()
            ads.append(pltpu.async_copy(
                q1.at[i1a_v.at[b]], rowsa_v.at[b], sem_a, add=True))
            ads.append(pltpu.async_copy(
                q1.at[i1b_v.at[b]], rowsb_v.at[b], sem_a, add=True))
        sts = []
        for b in range(K):
            ads[2 * b].wait()
            ads[2 * b + 1].wait()
            sts.append(pltpu.async_copy(
                rowsa_v.at[b],
                out_hbm.at[pl.ds(start + b * CH, CH), pl.ds(0, W)], sem_s))
            sts.append(pltpu.async_copy(
                rowsb_v.at[b],
                out_hbm.at[pl.ds(start + b * CH, CH), pl.ds(W, W)], sem_s))
        for d in sts:
            d.wait()


@functools.lru_cache(maxsize=None)
def _sc_gather_fn():
    return pl.kernel(
        _sc_gather_body,
        out_type=jax.ShapeDtypeStruct((HP, W2), jnp.float32),
        mesh=_sc_mesh(),
        scratch_types=[
            pltpu.VMEM((K, CH), jnp.int32),
            pltpu.VMEM((K, CH), jnp.int32),
            pltpu.VMEM((K, CH), jnp.int32),
            pltpu.VMEM((K, CH), jnp.int32),
            pltpu.VMEM((K, CH, W), jnp.float32),
            pltpu.VMEM((K, CH, W), jnp.float32),
            pltpu.SemaphoreType.DMA,
            pltpu.SemaphoreType.DMA,
            pltpu.SemaphoreType.DMA,
        ],
        compiler_params=pltpu.CompilerParams(use_tc_tiling_on_sc=False),
    )


def _sc_gather(q0, q1, i0a, i0b, i1a, i1b):
    return _sc_gather_fn()(q0, q1, i0a, i0b, i1a, i1b)


def _edge_enc_body(efa, efb, ewi, ebi, ewa, eba, ewb, ebb, ewo, ebo,
                   pw1, out):
    ef = jnp.concatenate([efa[...], efb[...]], axis=1)
    e_enc = _mlpb(ef, ewi[...], ebi[...], ewa[...], eba[...],
                  ewb[...], ebb[...], ewo[...], ebo[...])
    out[...] = _dotb(e_enc, pw1[...])


def _edge_encoder(ef, ws):
    nblk = HALF // TILE_P
    in_specs = [
        pl.BlockSpec((TILE_P, D_EDGE), lambda i: (i, 0)),
        pl.BlockSpec((TILE_P, D_EDGE), lambda i, _n=nblk: (i + _n, 0)),
    ]
    in_specs += [_full(w.shape) for w in ws]
    return pl.pallas_call(
        _edge_enc_body,
        grid=(nblk,),
        in_specs=in_specs,
        out_specs=pl.BlockSpec((TILE_P, W2), lambda i: (i, 0)),
        out_shape=jax.ShapeDtypeStruct((HP, W2), jnp.float32),
    )(ef, ef, *ws)


def _edge_body(ee1, g, pb, pwa, pba, pwb, pbb, pwo, pbo, out):
    h = jnp.maximum(ee1[...] + g[...] + pb[...], 0.0)
    h1 = jnp.maximum(_dotb(h, pwa[...]) + pba[...], 0.0)
    h2 = jnp.maximum(_dotb(h1, pwb[...]) + pbb[...], 0.0)
    h = h + h2
    out[...] = _dotb(h, pwo[...]) + pbo[...]


def _edge_mlps(ee1, gsum, ws):
    nblk = HALF // TILE_P
    in_specs = [
        pl.BlockSpec((TILE_P, W2), lambda i: (i, 0)),
        pl.BlockSpec((TILE_P, W2), lambda i: (i, 0)),
    ]
    in_specs += [_full(w.shape) for w in ws]
    return pl.pallas_call(
        _edge_body,
        grid=(nblk,),
        in_specs=in_specs,
        out_specs=pl.BlockSpec((TILE_P, W2), lambda i: (i, 0)),
        out_shape=jax.ShapeDtypeStruct((HP, W2), jnp.float32),
    )(ee1, gsum, *ws)


def _sc_scatter_body(eproc_hbm, ia_hbm, ib_hbm, zeros_hbm, out_hbm,
                     ia_v, ib_v, rowsa_v, rowsb_v, agg_sh, sem_l, sem_w):
    cid = lax.axis_index("c")
    sid = lax.axis_index("s")
    wid = sid * NC + cid

    pltpu.sync_copy(zeros_hbm.at[pl.ds(sid * NPT, NPT)],
                    agg_sh.at[pl.ds(sid * NPT, NPT)])
    plsc.subcore_barrier()

    crow0 = wid * P_CH

    @pl.loop(0, P_BLK)
    def _(g):
        row0 = crow0 + g * K
        start = row0 * CH
        pltpu.sync_copy(ia_hbm.at[pl.ds(row0, K)], ia_v)
        pltpu.sync_copy(ib_hbm.at[pl.ds(row0, K)], ib_v)
        lds = []
        for b in range(K):
            lds.append(pltpu.async_copy(
                eproc_hbm.at[pl.ds(start + b * CH, CH), pl.ds(0, W)],
                rowsa_v.at[b], sem_l))
            lds.append(pltpu.async_copy(
                eproc_hbm.at[pl.ds(start + b * CH, CH), pl.ds(W, W)],
                rowsb_v.at[b], sem_l))
        scs = []
        for b in range(K):
            lds[2 * b].wait()
            lds[2 * b + 1].wait()
            scs.append(pltpu.async_copy(
                rowsa_v.at[b], agg_sh.at[ia_v.at[b]], sem_w, add=True))
            scs.append(pltpu.async_copy(
                rowsb_v.at[b], agg_sh.at[ib_v.at[b]], sem_w, add=True))
        for d in scs:
            d.wait()

    plsc.subcore_barrier()
    pltpu.sync_copy(agg_sh.at[pl.ds(sid * NPT, NPT)],
                    out_hbm.at[cid, pl.ds(sid * NPT, NPT)])


@functools.lru_cache(maxsize=None)
def _sc_scatter_fn():
    return pl.kernel(
        _sc_scatter_body,
        out_type=jax.ShapeDtypeStruct((NC, N_PAD, W), jnp.float32),
        mesh=_sc_mesh(),
        scratch_types=[
            pltpu.VMEM((K, CH), jnp.int32),
            pltpu.VMEM((K, CH), jnp.int32),
            pltpu.VMEM((K, CH, W), jnp.float32),
            pltpu.VMEM((K, CH, W), jnp.float32),
            pltpu.VMEM_SHARED((N_PAD, W), jnp.float32),
            pltpu.SemaphoreType.DMA,
            pltpu.SemaphoreType.DMA,
        ],
        compiler_params=pltpu.CompilerParams(use_tc_tiling_on_sc=False),
    )


def _sc_scatter(e_proc, ia, ib, zeros):
    return _sc_scatter_fn()(e_proc, ia, ib, zeros)


def _node_body(nf, ne, a0, a1, nw1, nw2, nb, nwa, nba, nwb, nbb, nwo, nbo,
               dwi, dbi, dwa, dba, dwb, dbb, dwo, dbo, out):
    agg = a0[0] + a1[0]
    h = jnp.maximum(_dot(ne[...], nw1[...]) + _dot(agg, nw2[...]) + nb[...], 0.0)
    h1 = jnp.maximum(_dot(h, nwa[...]) + nba[...], 0.0)
    h2 = jnp.maximum(_dot(h1, nwb[...]) + nbb[...], 0.0)
    h = h + h2
    n_proc = _dot(h, nwo[...]) + nbo[...]
    dec = _mlp(n_proc, dwi[...], dbi[...], dwa[...], dba[...],
               dwb[...], dbb[...], dwo[...], dbo[...])
    out[...] = nf[...] + dec


def _node_mlps(nf, n_enc, aggp, ws):
    in_specs = [
        pl.BlockSpec((TILE_N, D_NODE), lambda i: (i, 0)),
        pl.BlockSpec((TILE_N, W), lambda i: (i, 0)),
        pl.BlockSpec((1, TILE_N, W), lambda i: (0, i, 0)),
        pl.BlockSpec((1, TILE_N, W), lambda i: (1, i, 0)),
    ]
    in_specs += [_full(w.shape) for w in ws]
    return pl.pallas_call(
        _node_body,
        grid=(N // TILE_N,),
        in_specs=in_specs,
        out_specs=pl.BlockSpec((TILE_N, D_NODE), lambda i: (i, 0)),
        out_shape=jax.ShapeDtypeStruct((N, D_NODE), jnp.float32),
    )(nf, n_enc, aggp, aggp, *ws)


def _bd(w):
    a, b = w.shape
    z = jnp.zeros((2 * a, 2 * b), w.dtype)
    return z.at[:a, :b].set(w).at[a:, b:].set(w)


def _b2(b):
    return jnp.concatenate([b, b])[None, :]


def _mlp_ws(p):
    blk = p["blocks"][0]
    return (p["in"]["W"], p["in"]["b"][None, :],
            blk["a"]["W"], blk["a"]["b"][None, :],
            blk["b"]["W"], blk["b"]["b"][None, :],
            p["out"]["W"], p["out"]["b"][None, :])


def _mlp_ws_bd(p):
    blk = p["blocks"][0]
    return (_bd(p["in"]["W"]), _b2(p["in"]["b"]),
            _bd(blk["a"]["W"]), _b2(blk["a"]["b"]),
            _bd(blk["b"]["W"]), _b2(blk["b"]["b"]),
            _bd(p["out"]["W"]), _b2(p["out"]["b"]))


def _half_idx(v, fill):
    a = jnp.pad(v[:HALF], (0, HP - HALF), constant_values=fill)
    b = jnp.pad(v[HALF:], (0, HP - HALF), constant_values=fill)
    return a.reshape(NIB, CH), b.reshape(NIB, CH)


def _run_one(e, nf, ef, params):
    ne_ws = _mlp_ws(params["node_encoder"])
    ee_ws = _mlp_ws_bd(params["edge_encoder"])
    ep = params["edge_processor"]
    ep_in = ep["in"]["W"]
    ep_ws = (_b2(ep["in"]["b"]),
             _bd(ep["blocks"][0]["a"]["W"]), _b2(ep["blocks"][0]["a"]["b"]),
             _bd(ep["blocks"][0]["b"]["W"]), _b2(ep["blocks"][0]["b"]["b"]),
             _bd(ep["out"]["W"]), _b2(ep["out"]["b"]))
    np_ = params["node_processor"]
    np_in = np_["in"]["W"]
    np_ws = (np_in[:W], np_in[W:], np_["in"]["b"][None, :],
             np_["blocks"][0]["a"]["W"], np_["blocks"][0]["a"]["b"][None, :],
             np_["blocks"][0]["b"]["W"], np_["blocks"][0]["b"]["b"][None, :],
             np_["out"]["W"], np_["out"]["b"][None, :])
    de_ws = _mlp_ws(params["decoder"])

    n_enc, q0, q1 = _node_encoder(nf, ne_ws + (ep_in[W:2 * W], ep_in[2 * W:]))

    i0a, i0b = _half_idx(e[0], 0)
    i1a, i1b = _half_idx(e[1], 0)
    gsum = _sc_gather(q0, q1, i0a, i0b, i1a, i1b)

    ee1 = _edge_encoder(ef, ee_ws + (_bd(ep_in[:W]),))
    e_proc = _edge_mlps(ee1, gsum, ep_ws)

    sa, sb = _half_idx(e[0], N)
    zeros = jnp.zeros((N_PAD, W), jnp.float32)
    aggp = _sc_scatter(e_proc, sa, sb, zeros)

    return _node_mlps(nf, n_enc, aggp, np_ws + de_ws)


def kernel(edges, node_features, edge_features, params):
    outs = [
        _run_one(edges[b], node_features[b], edge_features[b], params)
        for b in range(edges.shape[0])
    ]
    return jnp.stack(outs, axis=0)


# speedup vs baseline: 1.1050x; 1.1050x over previous
import functools

import jax
import jax.numpy as jnp
from jax import lax
from jax.experimental import pallas as pl
from jax.experimental.pallas import tpu as pltpu
from jax.experimental.pallas import tpu_sc as plsc

N = 10000
E = 320000
D_NODE = 128
D_EDGE = 16
W = 64
W2 = 2 * W

NC = 2
NS = 16
NW = NC * NS

HALF = E // 2
HP = 163840
TILE_N = 1000
TILE_P = 4000
N_PAD = 10240
CH = 64
K = 8

PPW = HP // NW
P_CH = PPW // CH
P_BLK = P_CH // K
NBLK = NW * P_BLK
B_C0 = 15
B_C1 = (NBLK - NS * B_C0) // NS
NPT = N_PAD // NS
NIB = HP // CH


@functools.lru_cache(maxsize=None)
def _sc_mesh():
    return plsc.VectorSubcoreMesh(
        core_axis_name="c", subcore_axis_name="s",
        num_cores=NC, num_subcores=NS)


def _dot(x, w):
    return lax.dot_general(x, w, (((1,), (0,)), ((), ())),
                           preferred_element_type=jnp.float32)


def _mlp(x, wi, bi, wa, ba, wb, bb, wo, bo):
    h = jnp.maximum(_dot(x, wi) + bi, 0.0)
    h1 = jnp.maximum(_dot(h, wa) + ba, 0.0)
    h2 = jnp.maximum(_dot(h1, wb) + bb, 0.0)
    h = h + h2
    return _dot(h, wo) + bo


def _full(shape):
    return pl.BlockSpec(shape, lambda i: (0,) * len(shape))


def _dotb(x, w):
    return lax.dot_general(x.astype(jnp.bfloat16), w.astype(jnp.bfloat16),
                           (((1,), (0,)), ((), ())),
                           preferred_element_type=jnp.float32)


def _mlpb(x, wi, bi, wa, ba, wb, bb, wo, bo):
    h = jnp.maximum(_dotb(x, wi) + bi, 0.0)
    h1 = jnp.maximum(_dotb(h, wa) + ba, 0.0)
    h2 = jnp.maximum(_dotb(h1, wb) + bb, 0.0)
    h = h + h2
    return _dotb(h, wo) + bo


def _node_enc_body(nf, wi, bi, wa, ba, wb, bb, wo, bo, pw2, pw3,
                   out, outq0, outq1):
    ne = _mlp(nf[...], wi[...], bi[...], wa[...], ba[...],
              wb[...], bb[...], wo[...], bo[...])
    out[...] = ne
    outq0[...] = _dot(ne, pw2[...])
    outq1[...] = _dot(ne, pw3[...])


def _node_encoder(nf, ws):
    in_specs = [pl.BlockSpec((TILE_N, D_NODE), lambda i: (i, 0))]
    in_specs += [_full(w.shape) for w in ws]
    ospec = pl.BlockSpec((TILE_N, W), lambda i: (i, 0))
    oshape = jax.ShapeDtypeStruct((N, W), jnp.float32)
    return pl.pallas_call(
        _node_enc_body,
        grid=(N // TILE_N,),
        in_specs=in_specs,
        out_specs=[ospec, ospec, ospec],
        out_shape=[oshape, oshape, oshape],
    )(nf, *ws)


def _sc_gather_body(q0, q1, i0a, i0b, i1a, i1b, out_hbm,
                    i0a_v, i0b_v, i1a_v, i1b_v, rowsa_v, rowsb_v,
                    sem_g, sem_a, sem_s):
    cid = lax.axis_index("c")
    sid = lax.axis_index("s")

    @pl.when(cid == 0)
    def _():
        _gather_loop(q0, q1, i0a, i0b, i1a, i1b, out_hbm, i0a_v, i0b_v,
                     i1a_v, i1b_v, rowsa_v, rowsb_v, sem_g, sem_a, sem_s,
                     sid * B_C0, B_C0)

    @pl.when(cid == 1)
    def _():
        _gather_loop(q0, q1, i0a, i0b, i1a, i1b, out_hbm, i0a_v, i0b_v,
                     i1a_v, i1b_v, rowsa_v, rowsb_v, sem_g, sem_a, sem_s,
                     NS * B_C0 + sid * B_C1, B_C1)


def _gather_loop(q0, q1, i0a, i0b, i1a, i1b, out_hbm,
                 i0a_v, i0b_v, i1a_v, i1b_v, rowsa_v, rowsb_v,
                 sem_g, sem_a, sem_s, blk0, nblk):
    @pl.loop(0, nblk)
    def _(g):
        row0 = (blk0 + g) * K
        start = row0 * CH
        pltpu.sync_copy(i0a.at[pl.ds(row0, K)], i0a_v)
        pltpu.sync_copy(i0b.at[pl.ds(row0, K)], i0b_v)
        pltpu.sync_copy(i1a.at[pl.ds(row0, K)], i1a_v)
        pltpu.sync_copy(i1b.at[pl.ds(row0, K)], i1b_v)
        gs = []
        for b in range(K):
            gs.append(pltpu.async_copy(
                q0.at[i0a_v.at[b]], rowsa_v.at[b], sem_g))
            gs.append(pltpu.async_copy(
                q0.at[i0b_v.at[b]], rowsb_v.at[b], sem_g))
        ads = []
        for b in range(K):
            gs[2 * b].wait()
            gs[2 * b + 1].wait()
            ads.append(pltpu.async_copy(
                q1.at[i1a_v.at[b]], rowsa_v.at[b], sem_a, add=True))
            ads.append(pltpu.async_copy(
                q1.at[i1b_v.at[b]], rowsb_v.at[b], sem_a, add=True))
        sts = []
        for b in range(K):
            ads[2 * b].wait()
            ads[2 * b + 1].wait()
            sts.append(pltpu.async_copy(
                rowsa_v.at[b],
                out_hbm.at[pl.ds(start + b * CH, CH), pl.ds(0, W)], sem_s))
            sts.append(pltpu.async_copy(
                rowsb_v.at[b],
                out_hbm.at[pl.ds(start + b * CH, CH), pl.ds(W, W)], sem_s))
        for d in sts:
            d.wait()


@functools.lru_cache(maxsize=None)
def _sc_gather_fn():
    return pl.kernel(
        _sc_gather_body,
        out_type=jax.ShapeDtypeStruct((HP, W2), jnp.float32),
        mesh=_sc_mesh(),
        scratch_types=[
            pltpu.VMEM((K, CH), jnp.int32),
            pltpu.VMEM((K, CH), jnp.int32),
            pltpu.VMEM((K, CH), jnp.int32),
            pltpu.VMEM((K, CH), jnp.int32),
            pltpu.VMEM((K, CH, W), jnp.float32),
            pltpu.VMEM((K, CH, W), jnp.float32),
            pltpu.SemaphoreType.DMA,
            pltpu.SemaphoreType.DMA,
            pltpu.SemaphoreType.DMA,
        ],
        compiler_params=pltpu.CompilerParams(use_tc_tiling_on_sc=False),
    )


def _sc_gather(q0, q1, i0a, i0b, i1a, i1b):
    return _sc_gather_fn()(q0, q1, i0a, i0b, i1a, i1b)


def _edge_enc_body(efa, efb, ewi, ebi, ewa, eba, ewb, ebb, ewo, ebo,
                   pw1, out):
    ef = jnp.concatenate([efa[...], efb[...]], axis=1)
    e_enc = _mlp(ef, ewi[...], ebi[...], ewa[...], eba[...],
                 ewb[...], ebb[...], ewo[...], ebo[...])
    out[...] = _dot(e_enc, pw1[...])


def _edge_encoder(ef, ws):
    nblk = HALF // TILE_P
    in_specs = [
        pl.BlockSpec((TILE_P, D_EDGE), lambda i: (i, 0)),
        pl.BlockSpec((TILE_P, D_EDGE), lambda i, _n=nblk: (i + _n, 0)),
    ]
    in_specs += [_full(w.shape) for w in ws]
    return pl.pallas_call(
        _edge_enc_body,
        grid=(nblk,),
        in_specs=in_specs,
        out_specs=pl.BlockSpec((TILE_P, W2), lambda i: (i, 0)),
        out_shape=jax.ShapeDtypeStruct((HP, W2), jnp.float32),
    )(ef, ef, *ws)


def _edge_body(ee1, g, pb, pwa, pba, pwb, pbb, pwo, pbo, out):
    h = jnp.maximum(ee1[...] + g[...] + pb[...], 0.0)
    h1 = jnp.maximum(_dot(h, pwa[...]) + pba[...], 0.0)
    h2 = jnp.maximum(_dot(h1, pwb[...]) + pbb[...], 0.0)
    h = h + h2
    out[...] = _dot(h, pwo[...]) + pbo[...]


def _edge_mlps(ee1, gsum, ws):
    nblk = HALF // TILE_P
    in_specs = [
        pl.BlockSpec((TILE_P, W2), lambda i: (i, 0)),
        pl.BlockSpec((TILE_P, W2), lambda i: (i, 0)),
    ]
    in_specs += [_full(w.shape) for w in ws]
    return pl.pallas_call(
        _edge_body,
        grid=(nblk,),
        in_specs=in_specs,
        out_specs=pl.BlockSpec((TILE_P, W2), lambda i: (i, 0)),
        out_shape=jax.ShapeDtypeStruct((HP, W2), jnp.float32),
    )(ee1, gsum, *ws)


def _sc_scatter_body(eproc_hbm, ia_hbm, ib_hbm, zeros_hbm, out_hbm,
                     ia_v, ib_v, rowsa_v, rowsb_v, agg_sh, sem_l, sem_w):
    cid = lax.axis_index("c")
    sid = lax.axis_index("s")
    wid = sid * NC + cid

    pltpu.sync_copy(zeros_hbm.at[pl.ds(sid * NPT, NPT)],
                    agg_sh.at[pl.ds(sid * NPT, NPT)])
    plsc.subcore_barrier()

    crow0 = wid * P_CH

    @pl.loop(0, P_BLK)
    def _(g):
        row0 = crow0 + g * K
        start = row0 * CH
        pltpu.sync_copy(ia_hbm.at[pl.ds(row0, K)], ia_v)
        pltpu.sync_copy(ib_hbm.at[pl.ds(row0, K)], ib_v)
        lds = []
        for b in range(K):
            lds.append(pltpu.async_copy(
                eproc_hbm.at[pl.ds(start + b * CH, CH), pl.ds(0, W)],
                rowsa_v.at[b], sem_l))
            lds.append(pltpu.async_copy(
                eproc_hbm.at[pl.ds(start + b * CH, CH), pl.ds(W, W)],
                rowsb_v.at[b], sem_l))
        scs = []
        for b in range(K):
            lds[2 * b].wait()
            lds[2 * b + 1].wait()
            scs.append(pltpu.async_copy(
                rowsa_v.at[b], agg_sh.at[ia_v.at[b]], sem_w, add=True))
            scs.append(pltpu.async_copy(
                rowsb_v.at[b], agg_sh.at[ib_v.at[b]], sem_w, add=True))
        for d in scs:
            d.wait()

    plsc.subcore_barrier()
    pltpu.sync_copy(agg_sh.at[pl.ds(sid * NPT, NPT)],
                    out_hbm.at[cid, pl.ds(sid * NPT, NPT)])


@functools.lru_cache(maxsize=None)
def _sc_scatter_fn():
    return pl.kernel(
        _sc_scatter_body,
        out_type=jax.ShapeDtypeStruct((NC, N_PAD, W), jnp.float32),
        mesh=_sc_mesh(),
        scratch_types=[
            pltpu.VMEM((K, CH), jnp.int32),
            pltpu.VMEM((K, CH), jnp.int32),
            pltpu.VMEM((K, CH, W), jnp.float32),
            pltpu.VMEM((K, CH, W), jnp.float32),
            pltpu.VMEM_SHARED((N_PAD, W), jnp.float32),
            pltpu.SemaphoreType.DMA,
            pltpu.SemaphoreType.DMA,
        ],
        compiler_params=pltpu.CompilerParams(use_tc_tiling_on_sc=False),
    )


def _sc_scatter(e_proc, ia, ib, zeros):
    return _sc_scatter_fn()(e_proc, ia, ib, zeros)


def _node_body(nf, ne, a0, a1, nw1, nw2, nb, nwa, nba, nwb, nbb, nwo, nbo,
               dwi, dbi, dwa, dba, dwb, dbb, dwo, dbo, out):
    agg = a0[0] + a1[0]
    h = jnp.maximum(_dot(ne[...], nw1[...]) + _dot(agg, nw2[...]) + nb[...], 0.0)
    h1 = jnp.maximum(_dot(h, nwa[...]) + nba[...], 0.0)
    h2 = jnp.maximum(_dot(h1, nwb[...]) + nbb[...], 0.0)
    h = h + h2
    n_proc = _dot(h, nwo[...]) + nbo[...]
    dec = _mlp(n_proc, dwi[...], dbi[...], dwa[...], dba[...],
               dwb[...], dbb[...], dwo[...], dbo[...])
    out[...] = nf[...] + dec


def _node_mlps(nf, n_enc, aggp, ws):
    in_specs = [
        pl.BlockSpec((TILE_N, D_NODE), lambda i: (i, 0)),
        pl.BlockSpec((TILE_N, W), lambda i: (i, 0)),
        pl.BlockSpec((1, TILE_N, W), lambda i: (0, i, 0)),
        pl.BlockSpec((1, TILE_N, W), lambda i: (1, i, 0)),
    ]
    in_specs += [_full(w.shape) for w in ws]
    return pl.pallas_call(
        _node_body,
        grid=(N // TILE_N,),
        in_specs=in_specs,
        out_specs=pl.BlockSpec((TILE_N, D_NODE), lambda i: (i, 0)),
        out_shape=jax.ShapeDtypeStruct((N, D_NODE), jnp.float32),
    )(nf, n_enc, aggp, aggp, *ws)


def _bd(w):
    a, b = w.shape
    z = jnp.zeros((2 * a, 2 * b), w.dtype)
    return z.at[:a, :b].set(w).at[a:, b:].set(w)


def _b2(b):
    return jnp.concatenate([b, b])[None, :]


def _mlp_ws(p):
    blk = p["blocks"][0]
    return (p["in"]["W"], p["in"]["b"][None, :],
            blk["a"]["W"], blk["a"]["b"][None, :],
            blk["b"]["W"], blk["b"]["b"][None, :],
            p["out"]["W"], p["out"]["b"][None, :])


def _mlp_ws_bd(p):
    blk = p["blocks"][0]
    return (_bd(p["in"]["W"]), _b2(p["in"]["b"]),
            _bd(blk["a"]["W"]), _b2(blk["a"]["b"]),
            _bd(blk["b"]["W"]), _b2(blk["b"]["b"]),
            _bd(p["out"]["W"]), _b2(p["out"]["b"]))


def _half_idx(v, fill):
    a = jnp.pad(v[:HALF], (0, HP - HALF), constant_values=fill)
    b = jnp.pad(v[HALF:], (0, HP - HALF), constant_values=fill)
    return a.reshape(NIB, CH), b.reshape(NIB, CH)


def _run_one(e, nf, ef, params):
    ne_ws = _mlp_ws(params["node_encoder"])
    ee_ws = _mlp_ws_bd(params["edge_encoder"])
    ep = params["edge_processor"]
    ep_in = ep["in"]["W"]
    ep_ws = (_b2(ep["in"]["b"]),
             _bd(ep["blocks"][0]["a"]["W"]), _b2(ep["blocks"][0]["a"]["b"]),
             _bd(ep["blocks"][0]["b"]["W"]), _b2(ep["blocks"][0]["b"]["b"]),
             _bd(ep["out"]["W"]), _b2(ep["out"]["b"]))
    np_ = params["node_processor"]
    np_in = np_["in"]["W"]
    np_ws = (np_in[:W], np_in[W:], np_["in"]["b"][None, :],
             np_["blocks"][0]["a"]["W"], np_["blocks"][0]["a"]["b"][None, :],
             np_["blocks"][0]["b"]["W"], np_["blocks"][0]["b"]["b"][None, :],
             np_["out"]["W"], np_["out"]["b"][None, :])
    de_ws = _mlp_ws(params["decoder"])

    n_enc, q0, q1 = _node_encoder(nf, ne_ws + (ep_in[W:2 * W], ep_in[2 * W:]))

    i0a, i0b = _half_idx(e[0], 0)
    i1a, i1b = _half_idx(e[1], 0)
    gsum = _sc_gather(q0, q1, i0a, i0b, i1a, i1b)

    ee1 = _edge_encoder(ef, ee_ws + (_bd(ep_in[:W]),))
    e_proc = _edge_mlps(ee1, gsum, ep_ws)

    sa, sb = _half_idx(e[0], N)
    zeros = jnp.zeros((N_PAD, W), jnp.float32)
    aggp = _sc_scatter(e_proc, sa, sb, zeros)

    return _node_mlps(nf, n_enc, aggp, np_ws + de_ws)


def kernel(edges, node_features, edge_features, params):
    outs = [
        _run_one(edges[b], node_features[b], edge_features[b], params)
        for b in range(edges.shape[0])
    ]
    return jnp.stack(outs, axis=0)


# speedup vs baseline: 1.1303x; 1.0229x over previous
import functools

import jax
import jax.numpy as jnp
from jax import lax
from jax.experimental import pallas as pl
from jax.experimental.pallas import tpu as pltpu
from jax.experimental.pallas import tpu_sc as plsc

N = 10000
E = 320000
D_NODE = 128
D_EDGE = 16
W = 64
W2 = 2 * W

NC = 2
NS = 16
NW = NC * NS

HALF = E // 2
HP = 163840
TILE_N = 1000
TILE_P = 8000
N_PAD = 10240
CH = 64
K = 8

PPW = HP // NW
P_CH = PPW // CH
P_BLK = P_CH // K
NBLK = NW * P_BLK
B_C0 = 15
B_C1 = (NBLK - NS * B_C0) // NS
NPT = N_PAD // NS
NIB = HP // CH


@functools.lru_cache(maxsize=None)
def _sc_mesh():
    return plsc.VectorSubcoreMesh(
        core_axis_name="c", subcore_axis_name="s",
        num_cores=NC, num_subcores=NS)


def _dot(x, w):
    return lax.dot_general(x, w, (((1,), (0,)), ((), ())),
                           preferred_element_type=jnp.float32)


def _mlp(x, wi, bi, wa, ba, wb, bb, wo, bo):
    h = jnp.maximum(_dot(x, wi) + bi, 0.0)
    h1 = jnp.maximum(_dot(h, wa) + ba, 0.0)
    h2 = jnp.maximum(_dot(h1, wb) + bb, 0.0)
    h = h + h2
    return _dot(h, wo) + bo


def _full(shape):
    return pl.BlockSpec(shape, lambda i: (0,) * len(shape))


def _dotb(x, w):
    return lax.dot_general(x.astype(jnp.bfloat16), w.astype(jnp.bfloat16),
                           (((1,), (0,)), ((), ())),
                           preferred_element_type=jnp.float32)


def _mlpb(x, wi, bi, wa, ba, wb, bb, wo, bo):
    h = jnp.maximum(_dotb(x, wi) + bi, 0.0)
    h1 = jnp.maximum(_dotb(h, wa) + ba, 0.0)
    h2 = jnp.maximum(_dotb(h1, wb) + bb, 0.0)
    h = h + h2
    return _dotb(h, wo) + bo


def _node_enc_body(nf, wi, bi, wa, ba, wb, bb, wo, bo, pw2, pw3,
                   out, outq0, outq1):
    ne = _mlp(nf[...], wi[...], bi[...], wa[...], ba[...],
              wb[...], bb[...], wo[...], bo[...])
    out[...] = ne
    outq0[...] = _dot(ne, pw2[...])
    outq1[...] = _dot(ne, pw3[...])


def _node_encoder(nf, ws):
    in_specs = [pl.BlockSpec((TILE_N, D_NODE), lambda i: (i, 0))]
    in_specs += [_full(w.shape) for w in ws]
    ospec = pl.BlockSpec((TILE_N, W), lambda i: (i, 0))
    oshape = jax.ShapeDtypeStruct((N, W), jnp.float32)
    return pl.pallas_call(
        _node_enc_body,
        grid=(N // TILE_N,),
        in_specs=in_specs,
        out_specs=[ospec, ospec, ospec],
        out_shape=[oshape, oshape, oshape],
    )(nf, *ws)


def _sc_gather_body(q0, q1, i0a, i0b, i1a, i1b, out_hbm,
                    i0a_v, i0b_v, i1a_v, i1b_v, rowsa_v, rowsb_v,
                    sem_g, sem_a, sem_s):
    cid = lax.axis_index("c")
    sid = lax.axis_index("s")

    @pl.when(cid == 0)
    def _():
        _gather_loop(q0, q1, i0a, i0b, i1a, i1b, out_hbm, i0a_v, i0b_v,
                     i1a_v, i1b_v, rowsa_v, rowsb_v, sem_g, sem_a, sem_s,
                     sid * B_C0, B_C0)

    @pl.when(cid == 1)
    def _():
        _gather_loop(q0, q1, i0a, i0b, i1a, i1b, out_hbm, i0a_v, i0b_v,
                     i1a_v, i1b_v, rowsa_v, rowsb_v, sem_g, sem_a, sem_s,
                     NS * B_C0 + sid * B_C1, B_C1)


def _gather_loop(q0, q1, i0a, i0b, i1a, i1b, out_hbm,
                 i0a_v, i0b_v, i1a_v, i1b_v, rowsa_v, rowsb_v,
                 sem_g, sem_a, sem_s, blk0, nblk):
    @pl.loop(0, nblk)
    def _(g):
        row0 = (blk0 + g) * K
        start = row0 * CH
        pltpu.sync_copy(i0a.at[pl.ds(row0, K)], i0a_v)
        pltpu.sync_copy(i0b.at[pl.ds(row0, K)], i0b_v)
        pltpu.sync_copy(i1a.at[pl.ds(row0, K)], i1a_v)
        pltpu.sync_copy(i1b.at[pl.ds(row0, K)], i1b_v)
        gs = []
        for b in range(K):
            gs.append(pltpu.async_copy(
                q0.at[i0a_v.at[b]], rowsa_v.at[b], sem_g))
            gs.append(pltpu.async_copy(
                q0.at[i0b_v.at[b]], rowsb_v.at[b], sem_g))
        ads = []
        for b in range(K):
            gs[2 * b].wait()
            gs[2 * b + 1].wait()
            ads.append(pltpu.async_copy(
                q1.at[i1a_v.at[b]], rowsa_v.at[b], sem_a, add=True))
            ads.append(pltpu.async_copy(
                q1.at[i1b_v.at[b]], rowsb_v.at[b], sem_a, add=True))
        sts = []
        for b in range(K):
            ads[2 * b].wait()
            ads[2 * b + 1].wait()
            sts.append(pltpu.async_copy(
                rowsa_v.at[b],
                out_hbm.at[pl.ds(start + b * CH, CH), pl.ds(0, W)], sem_s))
            sts.append(pltpu.async_copy(
                rowsb_v.at[b],
                out_hbm.at[pl.ds(start + b * CH, CH), pl.ds(W, W)], sem_s))
        for d in sts:
            d.wait()


@functools.lru_cache(maxsize=None)
def _sc_gather_fn():
    return pl.kernel(
        _sc_gather_body,
        out_type=jax.ShapeDtypeStruct((HP, W2), jnp.float32),
        mesh=_sc_mesh(),
        scratch_types=[
            pltpu.VMEM((K, CH), jnp.int32),
            pltpu.VMEM((K, CH), jnp.int32),
            pltpu.VMEM((K, CH), jnp.int32),
            pltpu.VMEM((K, CH), jnp.int32),
            pltpu.VMEM((K, CH, W), jnp.float32),
            pltpu.VMEM((K, CH, W), jnp.float32),
            pltpu.SemaphoreType.DMA,
            pltpu.SemaphoreType.DMA,
            pltpu.SemaphoreType.DMA,
        ],
        compiler_params=pltpu.CompilerParams(use_tc_tiling_on_sc=False),
    )


def _sc_gather(q0, q1, i0a, i0b, i1a, i1b):
    return _sc_gather_fn()(q0, q1, i0a, i0b, i1a, i1b)


def _edge_enc_body(efa, efb, ewi, ebi, ewa, eba, ewb, ebb, ewo, ebo,
                   pw1, out):
    ef = jnp.concatenate([efa[...], efb[...]], axis=1)
    e_enc = _mlp(ef, ewi[...], ebi[...], ewa[...], eba[...],
                 ewb[...], ebb[...], ewo[...], ebo[...])
    out[...] = _dot(e_enc, pw1[...])


def _edge_encoder(ef, ws):
    nblk = HALF // TILE_P
    in_specs = [
        pl.BlockSpec((TILE_P, D_EDGE), lambda i: (i, 0)),
        pl.BlockSpec((TILE_P, D_EDGE), lambda i, _n=nblk: (i + _n, 0)),
    ]
    in_specs += [_full(w.shape) for w in ws]
    return pl.pallas_call(
        _edge_enc_body,
        grid=(nblk,),
        in_specs=in_specs,
        out_specs=pl.BlockSpec((TILE_P, W2), lambda i: (i, 0)),
        out_shape=jax.ShapeDtypeStruct((HP, W2), jnp.float32),
    )(ef, ef, *ws)


def _edge_body(ee1, g, pb, pwa, pba, pwb, pbb, pwo, pbo, out):
    h = jnp.maximum(ee1[...] + g[...] + pb[...], 0.0)
    h1 = jnp.maximum(_dot(h, pwa[...]) + pba[...], 0.0)
    h2 = jnp.maximum(_dot(h1, pwb[...]) + pbb[...], 0.0)
    h = h + h2
    out[...] = _dot(h, pwo[...]) + pbo[...]


def _edge_mlps(ee1, gsum, ws):
    nblk = HALF // TILE_P
    in_specs = [
        pl.BlockSpec((TILE_P, W2), lambda i: (i, 0)),
        pl.BlockSpec((TILE_P, W2), lambda i: (i, 0)),
    ]
    in_specs += [_full(w.shape) for w in ws]
    return pl.pallas_call(
        _edge_body,
        grid=(nblk,),
        in_specs=in_specs,
        out_specs=pl.BlockSpec((TILE_P, W2), lambda i: (i, 0)),
        out_shape=jax.ShapeDtypeStruct((HP, W2), jnp.float32),
    )(ee1, gsum, *ws)


def _sc_scatter_body(eproc_hbm, ia_hbm, ib_hbm, zeros_hbm, out_hbm,
                     ia_v, ib_v, rowsa_v, rowsb_v, agg_sh, sem_l, sem_w):
    cid = lax.axis_index("c")
    sid = lax.axis_index("s")
    wid = sid * NC + cid

    pltpu.sync_copy(zeros_hbm.at[pl.ds(sid * NPT, NPT)],
                    agg_sh.at[pl.ds(sid * NPT, NPT)])
    plsc.subcore_barrier()

    crow0 = wid * P_CH

    @pl.loop(0, P_BLK)
    def _(g):
        row0 = crow0 + g * K
        start = row0 * CH
        pltpu.sync_copy(ia_hbm.at[pl.ds(row0, K)], ia_v)
        pltpu.sync_copy(ib_hbm.at[pl.ds(row0, K)], ib_v)
        lds = []
        for b in range(K):
            lds.append(pltpu.async_copy(
                eproc_hbm.at[pl.ds(start + b * CH, CH), pl.ds(0, W)],
                rowsa_v.at[b], sem_l))
            lds.append(pltpu.async_copy(
                eproc_hbm.at[pl.ds(start + b * CH, CH), pl.ds(W, W)],
                rowsb_v.at[b], sem_l))
        scs = []
        for b in range(K):
            lds[2 * b].wait()
            lds[2 * b + 1].wait()
            scs.append(pltpu.async_copy(
                rowsa_v.at[b], agg_sh.at[ia_v.at[b]], sem_w, add=True))
            scs.append(pltpu.async_copy(
                rowsb_v.at[b], agg_sh.at[ib_v.at[b]], sem_w, add=True))
        for d in scs:
            d.wait()

    plsc.subcore_barrier()
    pltpu.sync_copy(agg_sh.at[pl.ds(sid * NPT, NPT)],
                    out_hbm.at[cid, pl.ds(sid * NPT, NPT)])


@functools.lru_cache(maxsize=None)
def _sc_scatter_fn():
    return pl.kernel(
        _sc_scatter_body,
        out_type=jax.ShapeDtypeStruct((NC, N_PAD, W), jnp.float32),
        mesh=_sc_mesh(),
        scratch_types=[
            pltpu.VMEM((K, CH), jnp.int32),
            pltpu.VMEM((K, CH), jnp.int32),
            pltpu.VMEM((K, CH, W), jnp.float32),
            pltpu.VMEM((K, CH, W), jnp.float32),
            pltpu.VMEM_SHARED((N_PAD, W), jnp.float32),
            pltpu.SemaphoreType.DMA,
            pltpu.SemaphoreType.DMA,
        ],
        compiler_params=pltpu.CompilerParams(use_tc_tiling_on_sc=False),
    )


def _sc_scatter(e_proc, ia, ib, zeros):
    return _sc_scatter_fn()(e_proc, ia, ib, zeros)


def _node_body(nf, ne, a0, a1, nw1, nw2, nb, nwa, nba, nwb, nbb, nwo, nbo,
               dwi, dbi, dwa, dba, dwb, dbb, dwo, dbo, out):
    agg = a0[0] + a1[0]
    h = jnp.maximum(_dot(ne[...], nw1[...]) + _dot(agg, nw2[...]) + nb[...], 0.0)
    h1 = jnp.maximum(_dot(h, nwa[...]) + nba[...], 0.0)
    h2 = jnp.maximum(_dot(h1, nwb[...]) + nbb[...], 0.0)
    h = h + h2
    n_proc = _dot(h, nwo[...]) + nbo[...]
    dec = _mlp(n_proc, dwi[...], dbi[...], dwa[...], dba[...],
               dwb[...], dbb[...], dwo[...], dbo[...])
    out[...] = nf[...] + dec


def _node_mlps(nf, n_enc, aggp, ws):
    in_specs = [
        pl.BlockSpec((TILE_N, D_NODE), lambda i: (i, 0)),
        pl.BlockSpec((TILE_N, W), lambda i: (i, 0)),
        pl.BlockSpec((1, TILE_N, W), lambda i: (0, i, 0)),
        pl.BlockSpec((1, TILE_N, W), lambda i: (1, i, 0)),
    ]
    in_specs += [_full(w.shape) for w in ws]
    return pl.pallas_call(
        _node_body,
        grid=(N // TILE_N,),
        in_specs=in_specs,
        out_specs=pl.BlockSpec((TILE_N, D_NODE), lambda i: (i, 0)),
        out_shape=jax.ShapeDtypeStruct((N, D_NODE), jnp.float32),
    )(nf, n_enc, aggp, aggp, *ws)


def _bd(w):
    a, b = w.shape
    z = jnp.zeros((2 * a, 2 * b), w.dtype)
    return z.at[:a, :b].set(w).at[a:, b:].set(w)


def _b2(b):
    return jnp.concatenate([b, b])[None, :]


def _mlp_ws(p):
    blk = p["blocks"][0]
    return (p["in"]["W"], p["in"]["b"][None, :],
            blk["a"]["W"], blk["a"]["b"][None, :],
            blk["b"]["W"], blk["b"]["b"][None, :],
            p["out"]["W"], p["out"]["b"][None, :])


def _mlp_ws_bd(p):
    blk = p["blocks"][0]
    return (_bd(p["in"]["W"]), _b2(p["in"]["b"]),
            _bd(blk["a"]["W"]), _b2(blk["a"]["b"]),
            _bd(blk["b"]["W"]), _b2(blk["b"]["b"]),
            _bd(p["out"]["W"]), _b2(p["out"]["b"]))


def _half_idx(v, fill):
    a = jnp.pad(v[:HALF], (0, HP - HALF), constant_values=fill)
    b = jnp.pad(v[HALF:], (0, HP - HALF), constant_values=fill)
    return a.reshape(NIB, CH), b.reshape(NIB, CH)


def _run_one(e, nf, ef, params):
    ne_ws = _mlp_ws(params["node_encoder"])
    ee_ws = _mlp_ws_bd(params["edge_encoder"])
    ep = params["edge_processor"]
    ep_in = ep["in"]["W"]
    ep_ws = (_b2(ep["in"]["b"]),
             _bd(ep["blocks"][0]["a"]["W"]), _b2(ep["blocks"][0]["a"]["b"]),
             _bd(ep["blocks"][0]["b"]["W"]), _b2(ep["blocks"][0]["b"]["b"]),
             _bd(ep["out"]["W"]), _b2(ep["out"]["b"]))
    np_ = params["node_processor"]
    np_in = np_["in"]["W"]
    np_ws = (np_in[:W], np_in[W:], np_["in"]["b"][None, :],
             np_["blocks"][0]["a"]["W"], np_["blocks"][0]["a"]["b"][None, :],
             np_["blocks"][0]["b"]["W"], np_["blocks"][0]["b"]["b"][None, :],
             np_["out"]["W"], np_["out"]["b"][None, :])
    de_ws = _mlp_ws(params["decoder"])

    n_enc, q0, q1 = _node_encoder(nf, ne_ws + (ep_in[W:2 * W], ep_in[2 * W:]))

    i0a, i0b = _half_idx(e[0], 0)
    i1a, i1b = _half_idx(e[1], 0)
    gsum = _sc_gather(q0, q1, i0a, i0b, i1a, i1b)

    ee1 = _edge_encoder(ef, ee_ws + (_bd(ep_in[:W]),))
    e_proc = _edge_mlps(ee1, gsum, ep_ws)

    sa, sb = _half_idx(e[0], N)
    zeros = jnp.zeros((N_PAD, W), jnp.float32)
    aggp = _sc_scatter(e_proc, sa, sb, zeros)

    return _node_mlps(nf, n_enc, aggp, np_ws + de_ws)


def kernel(edges, node_features, edge_features, params):
    outs = [
        _run_one(edges[b], node_features[b], edge_features[b], params)
        for b in range(edges.shape[0])
    ]
    return jnp.stack(outs, axis=0)


# speedup vs baseline: 1.1585x; 1.0249x over previous
import functools

import jax
import jax.numpy as jnp
from jax import lax
from jax.experimental import pallas as pl
from jax.experimental.pallas import tpu as pltpu
from jax.experimental.pallas import tpu_sc as plsc

N = 10000
E = 320000
D_NODE = 128
D_EDGE = 16
W = 64
W2 = 2 * W

NC = 2
NS = 16
NW = NC * NS

HALF = E // 2
HP = 163840
TILE_N = 2000
TILE_P = 10000
N_PAD = 10240
CH = 64
K = 8

PPW = HP // NW
P_CH = PPW // CH
P_BLK = P_CH // K
NBLK = NW * P_BLK
B_C0 = 15
B_C1 = (NBLK - NS * B_C0) // NS
NPT = N_PAD // NS
NIB = HP // CH


@functools.lru_cache(maxsize=None)
def _sc_mesh():
    return plsc.VectorSubcoreMesh(
        core_axis_name="c", subcore_axis_name="s",
        num_cores=NC, num_subcores=NS)


def _dot(x, w):
    return lax.dot_general(x, w, (((1,), (0,)), ((), ())),
                           preferred_element_type=jnp.float32)


def _mlp(x, wi, bi, wa, ba, wb, bb, wo, bo):
    h = jnp.maximum(_dot(x, wi) + bi, 0.0)
    h1 = jnp.maximum(_dot(h, wa) + ba, 0.0)
    h2 = jnp.maximum(_dot(h1, wb) + bb, 0.0)
    h = h + h2
    return _dot(h, wo) + bo


def _full(shape):
    return pl.BlockSpec(shape, lambda i: (0,) * len(shape))


def _dotb(x, w):
    return lax.dot_general(x.astype(jnp.bfloat16), w.astype(jnp.bfloat16),
                           (((1,), (0,)), ((), ())),
                           preferred_element_type=jnp.float32)


def _mlpb(x, wi, bi, wa, ba, wb, bb, wo, bo):
    h = jnp.maximum(_dotb(x, wi) + bi, 0.0)
    h1 = jnp.maximum(_dotb(h, wa) + ba, 0.0)
    h2 = jnp.maximum(_dotb(h1, wb) + bb, 0.0)
    h = h + h2
    return _dotb(h, wo) + bo


def _node_enc_body(nf, wi, bi, wa, ba, wb, bb, wo, bo, pw2, pw3,
                   out, outq0, outq1):
    ne = _mlp(nf[...], wi[...], bi[...], wa[...], ba[...],
              wb[...], bb[...], wo[...], bo[...])
    out[...] = ne
    outq0[...] = _dot(ne, pw2[...])
    outq1[...] = _dot(ne, pw3[...])


def _node_encoder(nf, ws):
    in_specs = [pl.BlockSpec((TILE_N, D_NODE), lambda i: (i, 0))]
    in_specs += [_full(w.shape) for w in ws]
    ospec = pl.BlockSpec((TILE_N, W), lambda i: (i, 0))
    oshape = jax.ShapeDtypeStruct((N, W), jnp.float32)
    return pl.pallas_call(
        _node_enc_body,
        grid=(N // TILE_N,),
        in_specs=in_specs,
        out_specs=[ospec, ospec, ospec],
        out_shape=[oshape, oshape, oshape],
    )(nf, *ws)


def _sc_gather_body(q0, q1, i0a, i0b, i1a, i1b, out_hbm,
                    i0a_v, i0b_v, i1a_v, i1b_v, rowsa_v, rowsb_v,
                    sem_g, sem_a, sem_s):
    cid = lax.axis_index("c")
    sid = lax.axis_index("s")

    @pl.when(cid == 0)
    def _():
        _gather_loop(q0, q1, i0a, i0b, i1a, i1b, out_hbm, i0a_v, i0b_v,
                     i1a_v, i1b_v, rowsa_v, rowsb_v, sem_g, sem_a, sem_s,
                     sid * B_C0, B_C0)

    @pl.when(cid == 1)
    def _():
        _gather_loop(q0, q1, i0a, i0b, i1a, i1b, out_hbm, i0a_v, i0b_v,
                     i1a_v, i1b_v, rowsa_v, rowsb_v, sem_g, sem_a, sem_s,
                     NS * B_C0 + sid * B_C1, B_C1)


def _gather_loop(q0, q1, i0a, i0b, i1a, i1b, out_hbm,
                 i0a_v, i0b_v, i1a_v, i1b_v, rowsa_v, rowsb_v,
                 sem_g, sem_a, sem_s, blk0, nblk):
    @pl.loop(0, nblk)
    def _(g):
        row0 = (blk0 + g) * K
        start = row0 * CH
        pltpu.sync_copy(i0a.at[pl.ds(row0, K)], i0a_v)
        pltpu.sync_copy(i0b.at[pl.ds(row0, K)], i0b_v)
        pltpu.sync_copy(i1a.at[pl.ds(row0, K)], i1a_v)
        pltpu.sync_copy(i1b.at[pl.ds(row0, K)], i1b_v)
        gs = []
        for b in range(K):
            gs.append(pltpu.async_copy(
                q0.at[i0a_v.at[b]], rowsa_v.at[b], sem_g))
            gs.append(pltpu.async_copy(
                q0.at[i0b_v.at[b]], rowsb_v.at[b], sem_g))
        ads = []
        for b in range(K):
            gs[2 * b].wait()
            gs[2 * b + 1].wait()
            ads.append(pltpu.async_copy(
                q1.at[i1a_v.at[b]], rowsa_v.at[b], sem_a, add=True))
            ads.append(pltpu.async_copy(
                q1.at[i1b_v.at[b]], rowsb_v.at[b], sem_a, add=True))
        sts = []
        for b in range(K):
            ads[2 * b].wait()
            ads[2 * b + 1].wait()
            sts.append(pltpu.async_copy(
                rowsa_v.at[b],
                out_hbm.at[pl.ds(start + b * CH, CH), pl.ds(0, W)], sem_s))
            sts.append(pltpu.async_copy(
                rowsb_v.at[b],
                out_hbm.at[pl.ds(start + b * CH, CH), pl.ds(W, W)], sem_s))
        for d in sts:
            d.wait()


@functools.lru_cache(maxsize=None)
def _sc_gather_fn():
    return pl.kernel(
        _sc_gather_body,
        out_type=jax.ShapeDtypeStruct((HP, W2), jnp.float32),
        mesh=_sc_mesh(),
        scratch_types=[
            pltpu.VMEM((K, CH), jnp.int32),
            pltpu.VMEM((K, CH), jnp.int32),
            pltpu.VMEM((K, CH), jnp.int32),
            pltpu.VMEM((K, CH), jnp.int32),
            pltpu.VMEM((K, CH, W), jnp.float32),
            pltpu.VMEM((K, CH, W), jnp.float32),
            pltpu.SemaphoreType.DMA,
            pltpu.SemaphoreType.DMA,
            pltpu.SemaphoreType.DMA,
        ],
        compiler_params=pltpu.CompilerParams(use_tc_tiling_on_sc=False),
    )


def _sc_gather(q0, q1, i0a, i0b, i1a, i1b):
    return _sc_gather_fn()(q0, q1, i0a, i0b, i1a, i1b)


def _edge_enc_body(efa, efb, ewi, ebi, ewa, eba, ewb, ebb, ewo, ebo,
                   pw1, out):
    ef = jnp.concatenate([efa[...], efb[...]], axis=1)
    e_enc = _mlp(ef, ewi[...], ebi[...], ewa[...], eba[...],
                 ewb[...], ebb[...], ewo[...], ebo[...])
    out[...] = _dot(e_enc, pw1[...])


def _edge_encoder(ef, ws):
    nblk = HALF // TILE_P
    in_specs = [
        pl.BlockSpec((TILE_P, D_EDGE), lambda i: (i, 0)),
        pl.BlockSpec((TILE_P, D_EDGE), lambda i, _n=nblk: (i + _n, 0)),
    ]
    in_specs += [_full(w.shape) for w in ws]
    return pl.pallas_call(
        _edge_enc_body,
        grid=(nblk,),
        in_specs=in_specs,
        out_specs=pl.BlockSpec((TILE_P, W2), lambda i: (i, 0)),
        out_shape=jax.ShapeDtypeStruct((HP, W2), jnp.float32),
    )(ef, ef, *ws)


def _edge_body(ee1, g, pb, pwa, pba, pwb, pbb, pwo, pbo, out):
    h = jnp.maximum(ee1[...] + g[...] + pb[...], 0.0)
    h1 = jnp.maximum(_dot(h, pwa[...]) + pba[...], 0.0)
    h2 = jnp.maximum(_dot(h1, pwb[...]) + pbb[...], 0.0)
    h = h + h2
    out[...] = _dot(h, pwo[...]) + pbo[...]


def _edge_mlps(ee1, gsum, ws):
    nblk = HALF // TILE_P
    in_specs = [
        pl.BlockSpec((TILE_P, W2), lambda i: (i, 0)),
        pl.BlockSpec((TILE_P, W2), lambda i: (i, 0)),
    ]
    in_specs += [_full(w.shape) for w in ws]
    return pl.pallas_call(
        _edge_body,
        grid=(nblk,),
        in_specs=in_specs,
        out_specs=pl.BlockSpec((TILE_P, W2), lambda i: (i, 0)),
        out_shape=jax.ShapeDtypeStruct((HP, W2), jnp.float32),
    )(ee1, gsum, *ws)


def _sc_scatter_body(eproc_hbm, ia_hbm, ib_hbm, zeros_hbm, out_hbm,
                     ia_v, ib_v, rowsa_v, rowsb_v, agg_sh, sem_l, sem_w):
    cid = lax.axis_index("c")
    sid = lax.axis_index("s")
    wid = sid * NC + cid

    pltpu.sync_copy(zeros_hbm.at[pl.ds(sid * NPT, NPT)],
                    agg_sh.at[pl.ds(sid * NPT, NPT)])
    plsc.subcore_barrier()

    crow0 = wid * P_CH

    @pl.loop(0, P_BLK)
    def _(g):
        row0 = crow0 + g * K
        start = row0 * CH
        pltpu.sync_copy(ia_hbm.at[pl.ds(row0, K)], ia_v)
        pltpu.sync_copy(ib_hbm.at[pl.ds(row0, K)], ib_v)
        lds = []
        for b in range(K):
            lds.append(pltpu.async_copy(
                eproc_hbm.at[pl.ds(start + b * CH, CH), pl.ds(0, W)],
                rowsa_v.at[b], sem_l))
            lds.append(pltpu.async_copy(
                eproc_hbm.at[pl.ds(start + b * CH, CH), pl.ds(W, W)],
                rowsb_v.at[b], sem_l))
        scs = []
        for b in range(K):
            lds[2 * b].wait()
            lds[2 * b + 1].wait()
            scs.append(pltpu.async_copy(
                rowsa_v.at[b], agg_sh.at[ia_v.at[b]], sem_w, add=True))
            scs.append(pltpu.async_copy(
                rowsb_v.at[b], agg_sh.at[ib_v.at[b]], sem_w, add=True))
        for d in scs:
            d.wait()

    plsc.subcore_barrier()
    pltpu.sync_copy(agg_sh.at[pl.ds(sid * NPT, NPT)],
                    out_hbm.at[cid, pl.ds(sid * NPT, NPT)])


@functools.lru_cache(maxsize=None)
def _sc_scatter_fn():
    return pl.kernel(
        _sc_scatter_body,
        out_type=jax.ShapeDtypeStruct((NC, N_PAD, W), jnp.float32),
        mesh=_sc_mesh(),
        scratch_types=[
            pltpu.VMEM((K, CH), jnp.int32),
            pltpu.VMEM((K, CH), jnp.int32),
            pltpu.VMEM((K, CH, W), jnp.float32),
            pltpu.VMEM((K, CH, W), jnp.float32),
            pltpu.VMEM_SHARED((N_PAD, W), jnp.float32),
            pltpu.SemaphoreType.DMA,
            pltpu.SemaphoreType.DMA,
        ],
        compiler_params=pltpu.CompilerParams(use_tc_tiling_on_sc=False),
    )


def _sc_scatter(e_proc, ia, ib, zeros):
    return _sc_scatter_fn()(e_proc, ia, ib, zeros)


def _node_body(nf, ne, a0, a1, nw1, nw2, nb, nwa, nba, nwb, nbb, nwo, nbo,
               dwi, dbi, dwa, dba, dwb, dbb, dwo, dbo, out):
    agg = a0[0] + a1[0]
    h = jnp.maximum(_dot(ne[...], nw1[...]) + _dot(agg, nw2[...]) + nb[...], 0.0)
    h1 = jnp.maximum(_dot(h, nwa[...]) + nba[...], 0.0)
    h2 = jnp.maximum(_dot(h1, nwb[...]) + nbb[...], 0.0)
    h = h + h2
    n_proc = _dot(h, nwo[...]) + nbo[...]
    dec = _mlp(n_proc, dwi[...], dbi[...], dwa[...], dba[...],
               dwb[...], dbb[...], dwo[...], dbo[...])
    out[...] = nf[...] + dec


def _node_mlps(nf, n_enc, aggp, ws):
    in_specs = [
        pl.BlockSpec((TILE_N, D_NODE), lambda i: (i, 0)),
        pl.BlockSpec((TILE_N, W), lambda i: (i, 0)),
        pl.BlockSpec((1, TILE_N, W), lambda i: (0, i, 0)),
        pl.BlockSpec((1, TILE_N, W), lambda i: (1, i, 0)),
    ]
    in_specs += [_full(w.shape) for w in ws]
    return pl.pallas_call(
        _node_body,
        grid=(N // TILE_N,),
        in_specs=in_specs,
        out_specs=pl.BlockSpec((TILE_N, D_NODE), lambda i: (i, 0)),
        out_shape=jax.ShapeDtypeStruct((N, D_NODE), jnp.float32),
    )(nf, n_enc, aggp, aggp, *ws)


def _bd(w):
    a, b = w.shape
    z = jnp.zeros((2 * a, 2 * b), w.dtype)
    return z.at[:a, :b].set(w).at[a:, b:].set(w)


def _b2(b):
    return jnp.concatenate([b, b])[None, :]


def _mlp_ws(p):
    blk = p["blocks"][0]
    return (p["in"]["W"], p["in"]["b"][None, :],
            blk["a"]["W"], blk["a"]["b"][None, :],
            blk["b"]["W"], blk["b"]["b"][None, :],
            p["out"]["W"], p["out"]["b"][None, :])


def _mlp_ws_bd(p):
    blk = p["blocks"][0]
    return (_bd(p["in"]["W"]), _b2(p["in"]["b"]),
            _bd(blk["a"]["W"]), _b2(blk["a"]["b"]),
            _bd(blk["b"]["W"]), _b2(blk["b"]["b"]),
            _bd(p["out"]["W"]), _b2(p["out"]["b"]))


def _half_idx(v, fill):
    a = jnp.pad(v[:HALF], (0, HP - HALF), constant_values=fill)
    b = jnp.pad(v[HALF:], (0, HP - HALF), constant_values=fill)
    return a.reshape(NIB, CH), b.reshape(NIB, CH)


def _run_one(e, nf, ef, params):
    ne_ws = _mlp_ws(params["node_encoder"])
    ee_ws = _mlp_ws_bd(params["edge_encoder"])
    ep = params["edge_processor"]
    ep_in = ep["in"]["W"]
    ep_ws = (_b2(ep["in"]["b"]),
             _bd(ep["blocks"][0]["a"]["W"]), _b2(ep["blocks"][0]["a"]["b"]),
             _bd(ep["blocks"][0]["b"]["W"]), _b2(ep["blocks"][0]["b"]["b"]),
             _bd(ep["out"]["W"]), _b2(ep["out"]["b"]))
    np_ = params["node_processor"]
    np_in = np_["in"]["W"]
    np_ws = (np_in[:W], np_in[W:], np_["in"]["b"][None, :],
             np_["blocks"][0]["a"]["W"], np_["blocks"][0]["a"]["b"][None, :],
             np_["blocks"][0]["b"]["W"], np_["blocks"][0]["b"]["b"][None, :],
             np_["out"]["W"], np_["out"]["b"][None, :])
    de_ws = _mlp_ws(params["decoder"])

    n_enc, q0, q1 = _node_encoder(nf, ne_ws + (ep_in[W:2 * W], ep_in[2 * W:]))

    i0a, i0b = _half_idx(e[0], 0)
    i1a, i1b = _half_idx(e[1], 0)
    gsum = _sc_gather(q0, q1, i0a, i0b, i1a, i1b)

    ee1 = _edge_encoder(ef, ee_ws + (_bd(ep_in[:W]),))
    e_proc = _edge_mlps(ee1, gsum, ep_ws)

    sa, sb = _half_idx(e[0], N)
    zeros = jnp.zeros((N_PAD, W), jnp.float32)
    aggp = _sc_scatter(e_proc, sa, sb, zeros)

    return _node_mlps(nf, n_enc, aggp, np_ws + de_ws)


def kernel(edges, node_features, edge_features, params):
    outs = [
        _run_one(edges[b], node_features[b], edge_features[b], params)
        for b in range(edges.shape[0])
    ]
    return jnp.stack(outs, axis=0)
